# baseline (device time: 45679 ns/iter reference)
import jax
import jax.numpy as jnp
from jax import lax
from jax.experimental import pallas as pl
from jax.experimental.pallas import tpu as pltpu

N_DEV = 4
SEG = 2
L, R, O = 0, 1, 2
COMB, EARLY, DIRECT = 0, 1, 2


def kernel(x, W1, W2):
    m_per, d = x.shape
    mh = m_per // 2
    sr = mh // SEG

    def body(x_ref, w1_ref, w2_ref, out_ref,
             xgA, pA, rinA, pownA, xgB, pB, rinB, pownB,
             agSA, agRA, rsSA, rsRA, agSB, agRB, rsSB, rsRB):
        my = lax.axis_index("i")
        left = lax.rem(my + N_DEV - 1, N_DEV)
        right = lax.rem(my + 1, N_DEV)

        barrier = pltpu.get_barrier_semaphore()
        for nbr in (left, right):
            pl.semaphore_signal(barrier, inc=1, device_id=(nbr,),
                                device_id_type=pl.DeviceIdType.MESH)
        pl.semaphore_wait(barrier, 2)

        rings = (
            dict(xg=xgA, p=pA, rin=rinA, pown=pownA, agS=agSA, agR=agRA,
                 rsS=rsSA, rsR=rsRA, fwd=right, bwd=left, off=0),
            dict(xg=xgB, p=pB, rin=rinB, pown=pownB, agS=agSB, agR=agRB,
                 rsS=rsSB, rsR=rsRB, fwd=left, bwd=right, off=mh),
        )

        def rows(g):
            return slice(g * sr, (g + 1) * sr)

        def xrows(r, g):
            return slice(r["off"] + g * sr, r["off"] + (g + 1) * sr)

        def rdma(src, dst, ssem, rsem, dev):
            return pltpu.make_async_remote_copy(
                src_ref=src, dst_ref=dst, send_sem=ssem, recv_sem=rsem,
                device_id=(dev,), device_id_type=pl.DeviceIdType.MESH)

        def f(xc):
            h1 = jnp.dot(xc, w1_ref[:, :],
                         preferred_element_type=jnp.float32)
            h1 = h1 * (1.0 / (1.0 + jnp.exp(-h1)))
            return jnp.dot(h1, w2_ref[:, :],
                           preferred_element_type=jnp.float32)

        sends = []

        def start(desc):
            desc.start()
            sends.append(desc)
            return desc

        own_f, own_b, fwdd, early, comb, direct = {}, {}, {}, {}, {}, {}


        for g in range(SEG):
            for ri, r in enumerate(rings):
                own_f[ri, g] = start(rdma(
                    x_ref.at[xrows(r, g), :], r["xg"].at[L, rows(g), :],
                    r["agS"].at[L, g], r["agR"].at[L, g], r["fwd"]))
        for ri, r in enumerate(rings):
            own_b[ri, 0] = start(rdma(
                x_ref.at[xrows(r, 0), :], r["xg"].at[R, rows(0), :],
                r["agS"].at[R, 0], r["agR"].at[R, 0], r["bwd"]))
        for ri, r in enumerate(rings):
            r["pown"][rows(0), :] = f(x_ref[xrows(r, 0), :])

        for ri, r in enumerate(rings):
            own_f[ri, 0].wait_recv()
            fwdd[ri, 0] = start(rdma(
                r["xg"].at[L, rows(0), :], r["xg"].at[O, rows(0), :],
                r["agS"].at[O, 0], r["agR"].at[O, 0], r["fwd"]))
        for ri, r in enumerate(rings):
            own_b[ri, 1] = start(rdma(
                x_ref.at[xrows(r, 1), :], r["xg"].at[R, rows(1), :],
                r["agS"].at[R, 1], r["agR"].at[R, 1], r["bwd"]))
        for ri, r in enumerate(rings):
            r["pown"][rows(1), :] = f(x_ref[xrows(r, 1), :])
        for ri, r in enumerate(rings):
            own_f[ri, 1].wait_recv()
            fwdd[ri, 1] = start(rdma(
                r["xg"].at[L, rows(1), :], r["xg"].at[O, rows(1), :],
                r["agS"].at[O, 1], r["agR"].at[O, 1], r["fwd"]))

        for g in range(SEG):
            for ri, r in enumerate(rings):
                r["p"][DIRECT, rows(g), :] = f(r["xg"][L, rows(g), :])
                direct[ri, g] = start(rdma(
                    r["p"].at[DIRECT, rows(g), :],
                    r["rin"].at[2, rows(g), :],
                    r["rsS"].at[2, g], r["rsR"].at[2, g], r["bwd"]))

        for g in range(SEG):
            for ri, r in enumerate(rings):
                own_b[ri, g].wait_recv()
                r["p"][COMB, rows(g), :] = f(r["xg"][R, rows(g), :])

        for g in range(SEG):
            for ri, r in enumerate(rings):
                fwdd[ri, g].wait_recv()
                r["p"][EARLY, rows(g), :] = f(r["xg"][O, rows(g), :])
                early[ri, g] = start(rdma(
                    r["p"].at[EARLY, rows(g), :],
                    r["rin"].at[0, rows(g), :],
                    r["rsS"].at[0, g], r["rsR"].at[0, g], r["fwd"]))

        for g in range(SEG):
            for ri, r in enumerate(rings):
                early[ri, g].wait_recv()
                r["p"][COMB, rows(g), :] = (
                    r["p"][COMB, rows(g), :] + r["rin"][0, rows(g), :])
                comb[ri, g] = start(rdma(
                    r["p"].at[COMB, rows(g), :],
                    r["rin"].at[1, rows(g), :],
                    r["rsS"].at[1, g], r["rsR"].at[1, g], r["fwd"]))

        for g in range(SEG):
            for ri, r in enumerate(rings):
                direct[ri, g].wait_recv()
                comb[ri, g].wait_recv()
                out_ref[xrows(r, g), :] = (
                    r["pown"][rows(g), :] + r["rin"][1, rows(g), :]
                    + r["rin"][2, rows(g), :])

        for desc in sends:
            desc.wait_send()

    buf3 = (3, mh, d)
    sem3 = pltpu.SemaphoreType.DMA((3, SEG))
    return pl.pallas_call(
        body,
        out_shape=jax.ShapeDtypeStruct((m_per, d), jnp.float32),
        in_specs=[pl.BlockSpec(memory_space=pltpu.VMEM)] * 3,
        out_specs=pl.BlockSpec(memory_space=pltpu.VMEM),
        scratch_shapes=[
            pltpu.VMEM(buf3, jnp.float32),
            pltpu.VMEM(buf3, jnp.float32),
            pltpu.VMEM(buf3, jnp.float32),
            pltpu.VMEM((mh, d), jnp.float32),
            pltpu.VMEM(buf3, jnp.float32),
            pltpu.VMEM(buf3, jnp.float32),
            pltpu.VMEM(buf3, jnp.float32),
            pltpu.VMEM((mh, d), jnp.float32),
            sem3, sem3, sem3, sem3,
            sem3, sem3, sem3, sem3,
        ],
        compiler_params=pltpu.CompilerParams(collective_id=0),
    )(x, W1, W2)
